# Initial kernel scaffold; baseline (speedup 1.0000x reference)
import jax
import jax.numpy as jnp
from jax.experimental import pallas as pl


def kernel(inputs, table):
    raise NotImplementedError("write your pallas kernel here")


# speedup vs baseline: 1.5538x; 1.5538x over previous
import functools

import jax
import jax.numpy as jnp
from jax import lax
from jax.experimental import pallas as pl
from jax.experimental.pallas import tpu as pltpu
from jax.experimental.pallas import tpu_sc as plsc


def _make_gather(n, v, d):
    info = plsc.get_sparse_core_info()
    nc, ns = info.num_cores, info.num_subcores
    nw = nc * ns
    assert n % nw == 0
    b_per_w = n // nw
    chunk = 832
    while b_per_w % chunk != 0:
        chunk //= 2
    nchunks = b_per_w // chunk

    mesh = plsc.VectorSubcoreMesh(core_axis_name="c", subcore_axis_name="s")

    @functools.partial(
        pl.kernel,
        mesh=mesh,
        compiler_params=pltpu.CompilerParams(use_tc_tiling_on_sc=False),
        out_type=jax.ShapeDtypeStruct((n, d), jnp.float32),
        scratch_types=[
            pltpu.VMEM((b_per_w,), jnp.int32),
            pltpu.VMEM((chunk, d), jnp.float32),
            pltpu.SemaphoreType.DMA,
        ],
    )
    def gather_kernel(table_hbm, idx_hbm, out_hbm, idx_v, rows_v, sem):
        wid = lax.axis_index("s") * nc + lax.axis_index("c")
        base = wid * b_per_w
        pltpu.sync_copy(idx_hbm.at[pl.ds(base, b_per_w)], idx_v)

        def body(g, carry):
            off = pl.multiple_of(g * chunk, 8)
            pltpu.async_copy(
                table_hbm.at[idx_v.at[pl.ds(off, chunk)]], rows_v, sem
            ).wait()
            pltpu.sync_copy(rows_v, out_hbm.at[pl.ds(base + off, chunk)])
            return carry

        lax.fori_loop(0, nchunks, body, 0)

    return gather_kernel


def kernel(inputs, table):
    b, f = inputs.shape
    v, d = table.shape
    n = b * f
    flat_idx = inputs.reshape(n).astype(jnp.int32)
    out = _make_gather(n, v, d)(table, flat_idx)
    return out.reshape(b, f, d)


# speedup vs baseline: 1.5675x; 1.0088x over previous
import functools

import jax
import jax.numpy as jnp
from jax import lax
from jax.experimental import pallas as pl
from jax.experimental.pallas import tpu as pltpu
from jax.experimental.pallas import tpu_sc as plsc


def _make_gather(n, v, d):
    info = plsc.get_sparse_core_info()
    nc, ns = info.num_cores, info.num_subcores
    nw = nc * ns
    assert n % nw == 0
    b_per_w = n // nw
    chunk = 1664
    while b_per_w % chunk != 0:
        chunk //= 2
    nchunks = b_per_w // chunk

    mesh = plsc.VectorSubcoreMesh(core_axis_name="c", subcore_axis_name="s")

    @functools.partial(
        pl.kernel,
        mesh=mesh,
        compiler_params=pltpu.CompilerParams(use_tc_tiling_on_sc=False),
        out_type=jax.ShapeDtypeStruct((n, d), jnp.float32),
        scratch_types=[
            pltpu.VMEM((b_per_w,), jnp.int32),
            pltpu.VMEM((chunk, d), jnp.float32),
            pltpu.VMEM((chunk, d), jnp.float32),
            pltpu.SemaphoreType.DMA,
            pltpu.SemaphoreType.DMA,
            pltpu.SemaphoreType.DMA,
            pltpu.SemaphoreType.DMA,
        ],
    )
    def gather_kernel(table_hbm, idx_hbm, out_hbm, idx_v,
                      rows0, rows1, gsem0, gsem1, osem0, osem1):
        wid = lax.axis_index("s") * nc + lax.axis_index("c")
        base = wid * b_per_w
        pltpu.sync_copy(idx_hbm.at[pl.ds(base, b_per_w)], idx_v)

        rows = [rows0, rows1]
        gsems = [gsem0, gsem1]
        osems = [osem0, osem1]
        g_desc = [None, None]
        o_desc = [None, None]

        def issue_gather(g):
            bb = g % 2
            g_desc[bb] = pltpu.async_copy(
                table_hbm.at[idx_v.at[pl.ds(g * chunk, chunk)]],
                rows[bb], gsems[bb])

        def issue_out(g):
            bb = g % 2
            o_desc[bb] = pltpu.async_copy(
                rows[bb], out_hbm.at[pl.ds(base + g * chunk, chunk)],
                osems[bb])

        issue_gather(0)
        for g in range(nchunks):
            bb = g % 2
            g_desc[bb].wait()
            if g >= 1:
                o_desc[1 - bb].wait()
            if g + 1 < nchunks:
                issue_gather(g + 1)
            issue_out(g)
        o_desc[(nchunks - 1) % 2].wait()

    return gather_kernel


def kernel(inputs, table):
    b, f = inputs.shape
    v, d = table.shape
    n = b * f
    flat_idx = inputs.reshape(n).astype(jnp.int32)
    out = _make_gather(n, v, d)(table, flat_idx)
    return out.reshape(b, f, d)
